# baseline (speedup 1.0000x reference)
import functools

import jax
import jax.numpy as jnp
from jax import lax
from jax.experimental import pallas as pl
from jax.experimental.pallas import tpu as pltpu
from jax.experimental.pallas import tpu_sc as plsc

EMB = 64
SCALE = 8.0

NC = 2
NS = 16
NW = NC * NS

IDXW = 128
R = 4
CH = R * IDXW


def _sc_embed(idx2d, table, n_rows):
    nrow = idx2d.shape[0]
    rows_per_w = nrow // NW
    nchunk = rows_per_w // R

    mesh = plsc.VectorSubcoreMesh(core_axis_name="c", subcore_axis_name="s")

    @functools.partial(
        pl.kernel,
        mesh=mesh,
        out_type=jax.ShapeDtypeStruct((n_rows, EMB), jnp.float32),
        scratch_types=[
            pltpu.VMEM((R, IDXW), jnp.int32),
            pltpu.VMEM((CH, EMB), jnp.float32),
            pltpu.SemaphoreType.DMA,
        ],
        compiler_params=pltpu.CompilerParams(use_tc_tiling_on_sc=False),
    )
    def k(idx_hbm, table_hbm, out_hbm, idx_v, rows_v, gsem):
        wid = lax.axis_index("s") * NC + lax.axis_index("c")
        row0 = wid * rows_per_w

        def chunk(g, carry):
            rb = row0 + g * R
            pltpu.sync_copy(idx_hbm.at[pl.ds(rb, R)], idx_v)
            cps = [
                pltpu.async_copy(
                    table_hbm.at[idx_v.at[j]],
                    rows_v.at[pl.ds(j * IDXW, IDXW)],
                    gsem,
                )
                for j in range(R)
            ]
            for c in cps:
                c.wait()

            def mul(t, acc):
                i0 = t * 8
                for r in range(8):
                    for l in range(EMB // 16):
                        sl = pl.ds(l * 16, 16)
                        rows_v[i0 + r, sl] = rows_v[i0 + r, sl] * SCALE
                return acc

            lax.fori_loop(0, CH // 8, mul, 0)
            pltpu.sync_copy(rows_v, out_hbm.at[pl.ds(rb * IDXW, CH)])
            return carry

        lax.fori_loop(0, nchunk, chunk, 0)

    return k(idx2d, table)


def kernel(tokens, table):
    b0, b1 = tokens.shape
    n_rows = b0 * b1
    idx2d = jnp.reshape(tokens.astype(jnp.int32), (n_rows // IDXW, IDXW))
    out = _sc_embed(idx2d, table, n_rows)
    return jnp.reshape(out, (b0, b1, EMB))


# speedup vs baseline: 1.0542x; 1.0542x over previous
import functools

import jax
import jax.numpy as jnp
from jax import lax
from jax.experimental import pallas as pl
from jax.experimental.pallas import tpu as pltpu
from jax.experimental.pallas import tpu_sc as plsc

EMB = 64
SCALE = 8.0

NC = 2
NS = 16
NW = NC * NS

IDXW = 128
R = 4
CH = R * IDXW


def _sc_embed(idx2d, table, n_rows):
    nrow = idx2d.shape[0]
    rows_per_w = nrow // NW
    nchunk = rows_per_w // R

    mesh = plsc.VectorSubcoreMesh(core_axis_name="c", subcore_axis_name="s")

    @functools.partial(
        pl.kernel,
        mesh=mesh,
        out_type=jax.ShapeDtypeStruct((n_rows, EMB), jnp.float32),
        scratch_types=[
            pltpu.VMEM((2, R, IDXW), jnp.int32),
            pltpu.VMEM((2, CH, EMB), jnp.float32),
            pltpu.SemaphoreType.DMA((2,)),
            pltpu.SemaphoreType.DMA((2,)),
            pltpu.SemaphoreType.DMA((2,)),
        ],
        compiler_params=pltpu.CompilerParams(use_tc_tiling_on_sc=False),
    )
    def k(idx_hbm, table_hbm, out_hbm, idx_v, rows_v, isem, gsem, ssem):
        wid = lax.axis_index("s") * NC + lax.axis_index("c")
        row0 = wid * rows_per_w

        def fire_gather(g, b):
            for j in range(R):
                pltpu.async_copy(
                    table_hbm.at[idx_v.at[b, j]],
                    rows_v.at[b, pl.ds(j * IDXW, IDXW)],
                    gsem.at[b],
                )

        def fire_idx(g, b):
            pltpu.async_copy(idx_hbm.at[pl.ds(row0 + g * R, R)],
                             idx_v.at[b], isem.at[b])

        def wait_idx(b):
            pltpu.make_async_copy(idx_hbm.at[pl.ds(0, R)],
                                  idx_v.at[b], isem.at[b]).wait()

        def wait_gather(b):
            pltpu.make_async_copy(out_hbm.at[pl.ds(0, CH)],
                                  rows_v.at[b], gsem.at[b]).wait()

        def fire_store(g, b):
            pltpu.async_copy(rows_v.at[b],
                             out_hbm.at[pl.ds((row0 + g * R) * IDXW, CH)],
                             ssem.at[b])

        def wait_store(b):
            pltpu.make_async_copy(rows_v.at[b],
                                  out_hbm.at[pl.ds(0, CH)], ssem.at[b]).wait()

        def mul(b):
            def body(t, acc):
                i0 = t * 8
                for r in range(8):
                    for l in range(EMB // 16):
                        sl = pl.ds(l * 16, 16)
                        rows_v[b, i0 + r, sl] = rows_v[b, i0 + r, sl] * SCALE
                return acc

            lax.fori_loop(0, CH // 8, body, 0)

        pltpu.sync_copy(idx_hbm.at[pl.ds(row0, R)], idx_v.at[0])
        fire_gather(0, 0)
        fire_idx(1, 1)

        def step(t, carry):
            for b in range(2):
                g = t * 2 + b
                wait_gather(b)
                mul(b)
                fire_store(g, b)

                @pl.when(g + 1 < nchunk)
                def _():
                    wait_idx(1 - b)

                    @pl.when(g >= 1)
                    def _():
                        wait_store(1 - b)

                    fire_gather(g + 1, 1 - b)

                    @pl.when(g + 2 < nchunk)
                    def _():
                        fire_idx(g + 2, b)

            return carry

        lax.fori_loop(0, nchunk // 2, step, 0)
        wait_store(0)
        wait_store(1)

    return k(idx2d, table)


def kernel(tokens, table):
    b0, b1 = tokens.shape
    n_rows = b0 * b1
    idx2d = jnp.reshape(tokens.astype(jnp.int32), (n_rows // IDXW, IDXW))
    out = _sc_embed(idx2d, table, n_rows)
    return jnp.reshape(out, (b0, b1, EMB))
